# Initial kernel scaffold; baseline (speedup 1.0000x reference)
import jax
import jax.numpy as jnp
from jax.experimental import pallas as pl


def kernel(word_id, shape_id, word_table, shape_table):
    raise NotImplementedError("write your pallas kernel here")


# speedup vs baseline: 1.2049x; 1.2049x over previous
import functools

import jax
import jax.numpy as jnp
from jax import lax
from jax.experimental import pallas as pl
from jax.experimental.pallas import tpu as pltpu
from jax.experimental.pallas import tpu_sc as plsc

WORD_DIM = 64
SHAPE_DIM = 16
OUT_DIM = WORD_DIM + SHAPE_DIM
PAIR_DIM = 2 * WORD_DIM

_NUM_CORES = 2
_NUM_SUBCORES = 16
_NUM_WORKERS = _NUM_CORES * _NUM_SUBCORES

_IDX_W = 128
_CHUNK = 256
_NIDX = _CHUNK // _IDX_W
_L = 16


def _make_body(num_chunks_total, shape_vocab):
    num_chunks = num_chunks_total // _NUM_WORKERS

    def body(whalf_hbm, p64_hbm, sidx_hbm, wtab_hbm, stab_hbm, out_hbm,
             whalf_v, p64_v, sidx_v, pair_v, comb_v, stab_v, sem):
        sid = lax.axis_index("s")
        wid = sid * _NUM_CORES + lax.axis_index("c")
        chunk0 = wid * num_chunks

        pltpu.sync_copy(stab_hbm, stab_v)
        iota = lax.iota(jnp.int32, _L)

        def chunk(i, carry):
            ci = chunk0 + i
            pltpu.sync_copy(whalf_hbm.at[ci], whalf_v)
            pltpu.sync_copy(p64_hbm.at[ci], p64_v)
            pltpu.sync_copy(sidx_hbm.at[ci], sidx_v)
            copies = [
                pltpu.async_copy(
                    wtab_hbm.at[whalf_v.at[j]],
                    pair_v.at[pl.ds(j * _IDX_W, _IDX_W)], sem)
                for j in range(_NIDX)
            ]
            for c in copies:
                c.wait()

            def select(t, carry2):
                rowv = t * _L + iota
                src = p64_v[pl.ds(t * _L, _L)]
                dst = rowv * OUT_DIM
                sv = sidx_v[pl.ds(t * _L, _L)] * SHAPE_DIM
                for c in range(WORD_DIM):
                    val = plsc.load_gather(pair_v, [rowv, src + c])
                    plsc.store_scatter(comb_v, [dst + c], val)
                dst = dst + WORD_DIM
                for c in range(SHAPE_DIM):
                    val = plsc.load_gather(stab_v, [sv + c])
                    plsc.store_scatter(comb_v, [dst + c], val)
                return carry2

            lax.fori_loop(0, _CHUNK // _L, select, 0)
            pltpu.sync_copy(comb_v, out_hbm.at[ci])
            return carry

        lax.fori_loop(0, num_chunks, chunk, 0)

    return body


@jax.jit
def kernel(word_id, shape_id, word_table, shape_table):
    b, s = word_id.shape
    n = b * s
    num_chunks_total = n // _CHUNK
    wvocab = word_table.shape[0]
    svocab = shape_table.shape[0]

    wi = word_id.astype(jnp.int32)
    whalf = (wi >> 1).reshape(num_chunks_total, _NIDX, _IDX_W)
    p64 = ((wi & 1) << 6).reshape(num_chunks_total, _CHUNK)
    sidx = shape_id.astype(jnp.int32).reshape(num_chunks_total, _CHUNK)
    wtab2 = word_table.reshape(wvocab // 2, PAIR_DIM)
    stab_flat = shape_table.reshape(svocab * SHAPE_DIM)

    call = functools.partial(
        pl.kernel,
        out_type=jax.ShapeDtypeStruct((num_chunks_total, _CHUNK * OUT_DIM),
                                      jnp.float32),
        mesh=plsc.VectorSubcoreMesh(core_axis_name="c", subcore_axis_name="s"),
        compiler_params=pltpu.CompilerParams(needs_layout_passes=False),
        scratch_types=[
            pltpu.VMEM((_NIDX, _IDX_W), jnp.int32),
            pltpu.VMEM((_CHUNK,), jnp.int32),
            pltpu.VMEM((_CHUNK,), jnp.int32),
            pltpu.VMEM((_CHUNK, PAIR_DIM), jnp.float32),
            pltpu.VMEM((_CHUNK * OUT_DIM,), jnp.float32),
            pltpu.VMEM((svocab * SHAPE_DIM,), jnp.float32),
            pltpu.SemaphoreType.DMA,
        ],
    )(_make_body(num_chunks_total, svocab))
    out = call(whalf, p64, sidx, wtab2, stab_flat)
    return out.reshape(b, s, OUT_DIM)


# speedup vs baseline: 1.3729x; 1.1394x over previous
import functools

import jax
import jax.numpy as jnp
from jax import lax
from jax.experimental import pallas as pl
from jax.experimental.pallas import tpu as pltpu
from jax.experimental.pallas import tpu_sc as plsc

WORD_DIM = 64
SHAPE_DIM = 16
OUT_DIM = WORD_DIM + SHAPE_DIM
PAIR_DIM = 2 * WORD_DIM

_NUM_CORES = 2
_NUM_SUBCORES = 16
_NUM_WORKERS = _NUM_CORES * _NUM_SUBCORES

_IDX_W = 128
_CHUNK = 256
_NIDX = _CHUNK // _IDX_W
_L = 16


def _make_body(num_chunks_total, shape_vocab):
    num_chunks = num_chunks_total // _NUM_WORKERS
    assert num_chunks % 2 == 0

    def body(whalf_hbm, p64_hbm, sidx_hbm, wtab_hbm, stab_hbm, out_hbm,
             wh0, wh1, p0, p1, s0, s1, pair0, pair1, comb0, comb1, stab_v,
             g0, g1, w0, w1):
        wh = (wh0, wh1)
        pp = (p0, p1)
        ss = (s0, s1)
        pair = (pair0, pair1)
        comb = (comb0, comb1)
        gsem = (g0, g1)
        wsem = (w0, w1)
        sid = lax.axis_index("s")
        wid = sid * _NUM_CORES + lax.axis_index("c")
        chunk0 = wid * num_chunks

        pltpu.sync_copy(stab_hbm, stab_v)
        iota = lax.iota(jnp.int32, _L)

        def fire(ci, b):
            pltpu.sync_copy(whalf_hbm.at[ci], wh[b])
            pltpu.async_copy(p64_hbm.at[ci], pp[b], gsem[b])
            pltpu.async_copy(sidx_hbm.at[ci], ss[b], gsem[b])
            for j in range(_NIDX):
                pltpu.async_copy(
                    wtab_hbm.at[wh[b].at[j]],
                    pair[b].at[pl.ds(j * _IDX_W, _IDX_W)], gsem[b])

        def gather_wait(ci, b):
            pltpu.make_async_copy(p64_hbm.at[ci], pp[b], gsem[b]).wait()
            pltpu.make_async_copy(sidx_hbm.at[ci], ss[b], gsem[b]).wait()
            for j in range(_NIDX):
                pltpu.make_async_copy(
                    wtab_hbm.at[wh[b].at[j]],
                    pair[b].at[pl.ds(j * _IDX_W, _IDX_W)], gsem[b]).wait()

        def write_wait(ci, b):
            pltpu.make_async_copy(comb[b], out_hbm.at[ci], wsem[b]).wait()

        def select(b):
            def step(t, carry2):
                rowv = t * _L + iota
                src = pp[b][pl.ds(t * _L, _L)]
                dst = rowv * OUT_DIM
                sv = ss[b][pl.ds(t * _L, _L)] * SHAPE_DIM
                for c in range(WORD_DIM):
                    val = plsc.load_gather(pair[b], [rowv, src + c])
                    plsc.store_scatter(comb[b], [dst + c], val)
                dst = dst + WORD_DIM
                for c in range(SHAPE_DIM):
                    val = plsc.load_gather(stab_v, [sv + c])
                    plsc.store_scatter(comb[b], [dst + c], val)
                return carry2

            lax.fori_loop(0, _CHUNK // _L, step, 0)

        fire(chunk0, 0)

        def pair_iter(g, carry):
            ci0 = chunk0 + 2 * g
            for b in range(2):
                ci = ci0 + b
                nxt = ci + 1

                @pl.when(nxt < chunk0 + num_chunks)
                def _():
                    fire(nxt, 1 - b)

                gather_wait(ci, b)

                @pl.when(g >= 1)
                def _():
                    write_wait(ci - 2, b)

                select(b)
                pltpu.async_copy(comb[b], out_hbm.at[ci], wsem[b])
            return carry

        lax.fori_loop(0, num_chunks // 2, pair_iter, 0)
        write_wait(chunk0 + num_chunks - 2, 0)
        write_wait(chunk0 + num_chunks - 1, 1)

    return body


@jax.jit
def kernel(word_id, shape_id, word_table, shape_table):
    b, s = word_id.shape
    n = b * s
    num_chunks_total = n // _CHUNK
    wvocab = word_table.shape[0]
    svocab = shape_table.shape[0]

    wi = word_id.astype(jnp.int32)
    whalf = (wi >> 1).reshape(num_chunks_total, _NIDX, _IDX_W)
    p64 = ((wi & 1) << 6).reshape(num_chunks_total, _CHUNK)
    sidx = shape_id.astype(jnp.int32).reshape(num_chunks_total, _CHUNK)
    wtab2 = word_table.reshape(wvocab // 2, PAIR_DIM)
    stab_flat = shape_table.reshape(svocab * SHAPE_DIM)

    call = functools.partial(
        pl.kernel,
        out_type=jax.ShapeDtypeStruct((num_chunks_total, _CHUNK * OUT_DIM),
                                      jnp.float32),
        mesh=plsc.VectorSubcoreMesh(core_axis_name="c", subcore_axis_name="s"),
        compiler_params=pltpu.CompilerParams(needs_layout_passes=False),
        scratch_types=[
            pltpu.VMEM((_NIDX, _IDX_W), jnp.int32),
            pltpu.VMEM((_NIDX, _IDX_W), jnp.int32),
            pltpu.VMEM((_CHUNK,), jnp.int32),
            pltpu.VMEM((_CHUNK,), jnp.int32),
            pltpu.VMEM((_CHUNK,), jnp.int32),
            pltpu.VMEM((_CHUNK,), jnp.int32),
            pltpu.VMEM((_CHUNK, PAIR_DIM), jnp.float32),
            pltpu.VMEM((_CHUNK, PAIR_DIM), jnp.float32),
            pltpu.VMEM((_CHUNK * OUT_DIM,), jnp.float32),
            pltpu.VMEM((_CHUNK * OUT_DIM,), jnp.float32),
            pltpu.VMEM((svocab * SHAPE_DIM,), jnp.float32),
            pltpu.SemaphoreType.DMA,
            pltpu.SemaphoreType.DMA,
            pltpu.SemaphoreType.DMA,
            pltpu.SemaphoreType.DMA,
        ],
    )(_make_body(num_chunks_total, svocab))
    out = call(whalf, p64, sidx, wtab2, stab_flat)
    return out.reshape(b, s, OUT_DIM)


# speedup vs baseline: 2.1369x; 1.5565x over previous
import functools

import jax
import jax.numpy as jnp
from jax import lax
from jax.experimental import pallas as pl
from jax.experimental.pallas import tpu as pltpu
from jax.experimental.pallas import tpu_sc as plsc

WORD_DIM = 64
SHAPE_DIM = 16
OUT_DIM = WORD_DIM + SHAPE_DIM
PAIR_DIM = 2 * WORD_DIM

_NUM_CORES = 2
_NUM_SUBCORES = 16
_NUM_WORKERS = _NUM_CORES * _NUM_SUBCORES

_IDX_W = 128
_CHUNK = 256
_NIDX = _CHUNK // _IDX_W
_L = 16


def _make_body(num_chunks_total, shape_vocab):
    num_chunks = num_chunks_total // _NUM_WORKERS
    assert num_chunks % 2 == 0

    def body(whalf_hbm, p64_hbm, sidx_hbm, wtab_hbm, stab_hbm, out_hbm,
             wh0, wh1, p0, p1, s0, s1, pair0, pair1, comb0, comb1, stab_v,
             g0, g1, w0, w1):
        wh = (wh0, wh1)
        pp = (p0, p1)
        ss = (s0, s1)
        pair = (pair0, pair1)
        comb = (comb0, comb1)
        gsem = (g0, g1)
        wsem = (w0, w1)
        sid = lax.axis_index("s")
        wid = sid * _NUM_CORES + lax.axis_index("c")
        chunk0 = wid * num_chunks

        pltpu.sync_copy(stab_hbm, stab_v)
        iota = lax.iota(jnp.int32, _L)

        def fire(ci, b):
            pltpu.sync_copy(whalf_hbm.at[ci], wh[b])
            pltpu.async_copy(p64_hbm.at[ci], pp[b], gsem[b])
            pltpu.async_copy(sidx_hbm.at[ci], ss[b], gsem[b])
            for j in range(_NIDX):
                pltpu.async_copy(
                    wtab_hbm.at[wh[b].at[j]],
                    pair[b].at[pl.ds(j * _IDX_W, _IDX_W)], gsem[b])

        def gather_wait(ci, b):
            pltpu.make_async_copy(p64_hbm.at[ci], pp[b], gsem[b]).wait()
            pltpu.make_async_copy(sidx_hbm.at[ci], ss[b], gsem[b]).wait()
            for j in range(_NIDX):
                pltpu.make_async_copy(
                    wtab_hbm.at[wh[b].at[j]],
                    pair[b].at[pl.ds(j * _IDX_W, _IDX_W)], gsem[b]).wait()

        def write_wait(ci, b):
            pltpu.make_async_copy(comb[b], out_hbm.at[ci], wsem[b]).wait()

        def select(b):
            def step(t, carry2):
                rowv = t * _L + iota
                src = pp[b][pl.ds(t * _L, _L)]
                dst = rowv * OUT_DIM
                sv = ss[b][pl.ds(t * _L, _L)] * SHAPE_DIM
                for c in range(WORD_DIM):
                    col = (iota + c) & (WORD_DIM - 1)
                    val = plsc.load_gather(pair[b], [rowv, src + col])
                    plsc.store_scatter(comb[b], [dst + col], val)
                dst = dst + WORD_DIM
                for c in range(SHAPE_DIM):
                    col = (iota + c) & (SHAPE_DIM - 1)
                    val = plsc.load_gather(stab_v, [sv + col])
                    plsc.store_scatter(comb[b], [dst + col], val)
                return carry2

            lax.fori_loop(0, _CHUNK // _L, step, 0)

        fire(chunk0, 0)

        def pair_iter(g, carry):
            ci0 = chunk0 + 2 * g
            for b in range(2):
                ci = ci0 + b
                nxt = ci + 1

                @pl.when(nxt < chunk0 + num_chunks)
                def _():
                    fire(nxt, 1 - b)

                gather_wait(ci, b)

                @pl.when(g >= 1)
                def _():
                    write_wait(ci - 2, b)

                select(b)
                pltpu.async_copy(comb[b], out_hbm.at[ci], wsem[b])
            return carry

        lax.fori_loop(0, num_chunks // 2, pair_iter, 0)
        write_wait(chunk0 + num_chunks - 2, 0)
        write_wait(chunk0 + num_chunks - 1, 1)

    return body


@jax.jit
def kernel(word_id, shape_id, word_table, shape_table):
    b, s = word_id.shape
    n = b * s
    num_chunks_total = n // _CHUNK
    wvocab = word_table.shape[0]
    svocab = shape_table.shape[0]

    wi = word_id.astype(jnp.int32)
    whalf = (wi >> 1).reshape(num_chunks_total, _NIDX, _IDX_W)
    p64 = ((wi & 1) << 6).reshape(num_chunks_total, _CHUNK)
    sidx = shape_id.astype(jnp.int32).reshape(num_chunks_total, _CHUNK)
    wtab2 = word_table.reshape(wvocab // 2, PAIR_DIM)
    stab_flat = shape_table.reshape(svocab * SHAPE_DIM)

    call = functools.partial(
        pl.kernel,
        out_type=jax.ShapeDtypeStruct((num_chunks_total, _CHUNK * OUT_DIM),
                                      jnp.float32),
        mesh=plsc.VectorSubcoreMesh(core_axis_name="c", subcore_axis_name="s"),
        compiler_params=pltpu.CompilerParams(needs_layout_passes=False),
        scratch_types=[
            pltpu.VMEM((_NIDX, _IDX_W), jnp.int32),
            pltpu.VMEM((_NIDX, _IDX_W), jnp.int32),
            pltpu.VMEM((_CHUNK,), jnp.int32),
            pltpu.VMEM((_CHUNK,), jnp.int32),
            pltpu.VMEM((_CHUNK,), jnp.int32),
            pltpu.VMEM((_CHUNK,), jnp.int32),
            pltpu.VMEM((_CHUNK, PAIR_DIM), jnp.float32),
            pltpu.VMEM((_CHUNK, PAIR_DIM), jnp.float32),
            pltpu.VMEM((_CHUNK * OUT_DIM,), jnp.float32),
            pltpu.VMEM((_CHUNK * OUT_DIM,), jnp.float32),
            pltpu.VMEM((svocab * SHAPE_DIM,), jnp.float32),
            pltpu.SemaphoreType.DMA,
            pltpu.SemaphoreType.DMA,
            pltpu.SemaphoreType.DMA,
            pltpu.SemaphoreType.DMA,
        ],
    )(_make_body(num_chunks_total, svocab))
    out = call(whalf, p64, sidx, wtab2, stab_flat)
    return out.reshape(b, s, OUT_DIM)


# speedup vs baseline: 2.3763x; 1.1120x over previous
import functools

import jax
import jax.numpy as jnp
from jax import lax
from jax.experimental import pallas as pl
from jax.experimental.pallas import tpu as pltpu
from jax.experimental.pallas import tpu_sc as plsc

WORD_DIM = 64
SHAPE_DIM = 16
OUT_DIM = WORD_DIM + SHAPE_DIM
PAIR_DIM = 2 * WORD_DIM

_NUM_CORES = 2
_NUM_SUBCORES = 16
_NUM_WORKERS = _NUM_CORES * _NUM_SUBCORES

_IDX_W = 128
_CHUNK = 256
_NIDX = _CHUNK // _IDX_W
_L = 16


def _make_body(num_chunks_total, shape_vocab):
    num_chunks = num_chunks_total // _NUM_WORKERS
    assert num_chunks % 2 == 0

    def body(whalf_hbm, p64_hbm, sidx_hbm, wtab_raw, stab_hbm, out_raw,
             wh0, wh1, p0, p1, s0, s1, pair0, pair1, comb0, stab_v,
             g0, g1, w0, w1):
        wtab_hbm = wtab_raw
        out_hbm = out_raw.reshape(num_chunks_total * _CHUNK, OUT_DIM)
        wh = (wh0, wh1)
        pp = (p0, p1)
        ss = (s0, s1)
        pair = (pair0, pair1)
        comb = (comb0, comb0)
        gsem = (g0, g1)
        wsem = (w0, w0)
        sid = lax.axis_index("s")
        wid = sid * _NUM_CORES + lax.axis_index("c")
        chunk0 = wid * num_chunks

        pltpu.sync_copy(stab_hbm, stab_v)
        iota = lax.iota(jnp.int32, _L)

        def fire(ci, b):
            pltpu.sync_copy(whalf_hbm.at[ci], wh[b])
            pltpu.async_copy(p64_hbm.at[ci], pp[b], gsem[b])
            pltpu.async_copy(sidx_hbm.at[ci], ss[b], gsem[b])
            for j in range(_NIDX):
                pltpu.async_copy(
                    wtab_hbm.at[wh[b].at[j]],
                    pair[b].at[pl.ds(j * _IDX_W, _IDX_W)], gsem[b])

        def gather_wait(ci, b):
            pltpu.make_async_copy(p64_hbm.at[ci], pp[b], gsem[b]).wait()
            pltpu.make_async_copy(sidx_hbm.at[ci], ss[b], gsem[b]).wait()
            for j in range(_NIDX):
                pltpu.make_async_copy(
                    wtab_hbm.at[wh[b].at[j]],
                    pair[b].at[pl.ds(j * _IDX_W, _IDX_W)], gsem[b]).wait()

        def write_wait(ci, b):
            pltpu.make_async_copy(
                comb[b], out_hbm.at[pl.ds(ci * _CHUNK, _CHUNK)],
                wsem[b]).wait()

        def select(b):
            def step(t, carry2):
                rowv = t * _L + iota
                src = pp[b][pl.ds(t * _L, _L)]
                sv = ss[b][pl.ds(t * _L, _L)] * SHAPE_DIM
                for c in range(WORD_DIM):
                    col = (iota + c) & (WORD_DIM - 1)
                    flat = src + col
                    val = plsc.load_gather(pair[b], [rowv, flat])
                    plsc.store_scatter(comb[b], [rowv, col], val)
                for c in range(SHAPE_DIM):
                    col = (iota + c) & (SHAPE_DIM - 1)
                    val = plsc.load_gather(stab_v, [sv + col])
                    plsc.store_scatter(comb[b], [rowv, WORD_DIM + col], val)
                return carry2

            lax.fori_loop(0, _CHUNK // _L, step, 0)

        fire(chunk0, 0)

        def pair_iter(g, carry):
            ci0 = chunk0 + 2 * g
            for b in range(2):
                ci = ci0 + b
                nxt = ci + 1

                @pl.when(nxt < chunk0 + num_chunks)
                def _():
                    fire(nxt, 1 - b)

                gather_wait(ci, b)

                @pl.when(ci > chunk0)
                def _():
                    write_wait(ci - 1, b)

                select(b)
                pltpu.async_copy(
                    comb[b], out_hbm.at[pl.ds(ci * _CHUNK, _CHUNK)], wsem[b])
            return carry

        lax.fori_loop(0, num_chunks // 2, pair_iter, 0)
        write_wait(chunk0 + num_chunks - 1, 1)

    return body


@jax.jit
def kernel(word_id, shape_id, word_table, shape_table):
    b, s = word_id.shape
    n = b * s
    num_chunks_total = n // _CHUNK
    wvocab = word_table.shape[0]
    svocab = shape_table.shape[0]

    wi = word_id.astype(jnp.int32)
    whalf = (wi >> 1).reshape(num_chunks_total, _NIDX, _IDX_W)
    p64 = ((wi & 1) << 6).reshape(num_chunks_total, _CHUNK)
    sidx = shape_id.astype(jnp.int32).reshape(num_chunks_total, _CHUNK)
    stab_flat = shape_table.reshape(svocab * SHAPE_DIM)
    wtab2 = word_table.reshape(wvocab // 2, PAIR_DIM)

    call = functools.partial(
        pl.kernel,
        out_type=jax.ShapeDtypeStruct((b, s, OUT_DIM), jnp.float32),
        mesh=plsc.VectorSubcoreMesh(core_axis_name="c", subcore_axis_name="s"),
        compiler_params=pltpu.CompilerParams(needs_layout_passes=False),
        scratch_types=[
            pltpu.VMEM((_NIDX, _IDX_W), jnp.int32),
            pltpu.VMEM((_NIDX, _IDX_W), jnp.int32),
            pltpu.VMEM((_CHUNK,), jnp.int32),
            pltpu.VMEM((_CHUNK,), jnp.int32),
            pltpu.VMEM((_CHUNK,), jnp.int32),
            pltpu.VMEM((_CHUNK,), jnp.int32),
            pltpu.VMEM((_CHUNK, PAIR_DIM), jnp.float32),
            pltpu.VMEM((_CHUNK, PAIR_DIM), jnp.float32),
            pltpu.VMEM((_CHUNK, OUT_DIM), jnp.float32),
            pltpu.VMEM((svocab * SHAPE_DIM,), jnp.float32),
            pltpu.SemaphoreType.DMA,
            pltpu.SemaphoreType.DMA,
            pltpu.SemaphoreType.DMA,
            pltpu.SemaphoreType.DMA,
        ],
    )(_make_body(num_chunks_total, svocab))
    return call(whalf, p64, sidx, wtab2, stab_flat)


# speedup vs baseline: 2.3768x; 1.0002x over previous
import functools

import jax
import jax.numpy as jnp
from jax import lax
from jax.experimental import pallas as pl
from jax.experimental.layout import Format, Layout
from jax.experimental.pallas import tpu as pltpu
from jax.experimental.pallas import tpu_sc as plsc

WORD_DIM = 64
SHAPE_DIM = 16
OUT_DIM = WORD_DIM + SHAPE_DIM
PAIR_DIM = 2 * WORD_DIM

_NUM_CORES = 2
_NUM_SUBCORES = 16
_NUM_WORKERS = _NUM_CORES * _NUM_SUBCORES

_IDX_W = 128
_CHUNK = 256
_NIDX = _CHUNK // _IDX_W
_L = 16


def _make_body(num_chunks_total, shape_vocab):
    num_chunks = num_chunks_total // _NUM_WORKERS
    assert num_chunks % 2 == 0

    def body(whalf_hbm, p64_hbm, sidx_hbm, wtab_raw, stab_hbm, out_raw,
             wh0, wh1, p0, p1, s0, s1, pair0, pair1, comb0, stab_v,
             g0, g1, w0, w1):
        wtab_hbm = wtab_raw
        out_hbm = out_raw.reshape(num_chunks_total * _CHUNK, OUT_DIM)
        wh = (wh0, wh1)
        pp = (p0, p1)
        ss = (s0, s1)
        pair = (pair0, pair1)
        comb = (comb0, comb0)
        gsem = (g0, g1)
        wsem = (w0, w0)
        sid = lax.axis_index("s")
        wid = sid * _NUM_CORES + lax.axis_index("c")
        chunk0 = wid * num_chunks

        pltpu.sync_copy(stab_hbm, stab_v)
        iota = lax.iota(jnp.int32, _L)

        def fire(ci, b):
            pltpu.sync_copy(whalf_hbm.at[ci], wh[b])
            pltpu.async_copy(p64_hbm.at[ci], pp[b], gsem[b])
            pltpu.async_copy(sidx_hbm.at[ci], ss[b], gsem[b])
            for j in range(_NIDX):
                pltpu.async_copy(
                    wtab_hbm.at[wh[b].at[j]],
                    pair[b].at[pl.ds(j * _IDX_W, _IDX_W)], gsem[b])

        def gather_wait(ci, b):
            pltpu.make_async_copy(p64_hbm.at[ci], pp[b], gsem[b]).wait()
            pltpu.make_async_copy(sidx_hbm.at[ci], ss[b], gsem[b]).wait()
            for j in range(_NIDX):
                pltpu.make_async_copy(
                    wtab_hbm.at[wh[b].at[j]],
                    pair[b].at[pl.ds(j * _IDX_W, _IDX_W)], gsem[b]).wait()

        def write_wait(ci, b):
            pltpu.make_async_copy(
                comb[b], out_hbm.at[pl.ds(ci * _CHUNK, _CHUNK)],
                wsem[b]).wait()

        def select(b):
            def step(t, carry2):
                rowv = t * _L + iota
                src = pp[b][pl.ds(t * _L, _L)]
                sv = ss[b][pl.ds(t * _L, _L)] * SHAPE_DIM
                for c in range(WORD_DIM):
                    col = (iota + c) & (WORD_DIM - 1)
                    flat = src + col
                    val = plsc.load_gather(pair[b], [rowv, flat])
                    plsc.store_scatter(comb[b], [rowv, col], val)
                for c in range(SHAPE_DIM):
                    col = (iota + c) & (SHAPE_DIM - 1)
                    val = plsc.load_gather(stab_v, [sv + col])
                    plsc.store_scatter(comb[b], [rowv, WORD_DIM + col], val)
                return carry2

            lax.fori_loop(0, _CHUNK // _L, step, 0)

        fire(chunk0, 0)

        def pair_iter(g, carry):
            ci0 = chunk0 + 2 * g
            for b in range(2):
                ci = ci0 + b
                nxt = ci + 1

                @pl.when(nxt < chunk0 + num_chunks)
                def _():
                    fire(nxt, 1 - b)

                gather_wait(ci, b)

                @pl.when(ci > chunk0)
                def _():
                    write_wait(ci - 1, b)

                select(b)
                pltpu.async_copy(
                    comb[b], out_hbm.at[pl.ds(ci * _CHUNK, _CHUNK)], wsem[b])
            return carry

        lax.fori_loop(0, num_chunks // 2, pair_iter, 0)
        write_wait(chunk0 + num_chunks - 1, 1)

    return body


@jax.jit
def kernel(word_id, shape_id, word_table, shape_table):
    b, s = word_id.shape
    n = b * s
    num_chunks_total = n // _CHUNK
    wvocab = word_table.shape[0]
    svocab = shape_table.shape[0]

    wi = word_id.astype(jnp.int32)
    whalf = (wi >> 1).reshape(num_chunks_total, _NIDX, _IDX_W)
    p64 = ((wi & 1) << 6).reshape(num_chunks_total, _CHUNK)
    sidx = shape_id.astype(jnp.int32).reshape(num_chunks_total, _CHUNK)
    stab_flat = shape_table.reshape(svocab * SHAPE_DIM)
    wtab2 = word_table.reshape(wvocab // 2, PAIR_DIM) * jnp.float32(1.0)

    call = functools.partial(
        pl.kernel,
        out_type=jax.ShapeDtypeStruct((b, s, OUT_DIM), jnp.float32),
        mesh=plsc.VectorSubcoreMesh(core_axis_name="c", subcore_axis_name="s"),
        compiler_params=pltpu.CompilerParams(needs_layout_passes=False),
        scratch_types=[
            pltpu.VMEM((_NIDX, _IDX_W), jnp.int32),
            pltpu.VMEM((_NIDX, _IDX_W), jnp.int32),
            pltpu.VMEM((_CHUNK,), jnp.int32),
            pltpu.VMEM((_CHUNK,), jnp.int32),
            pltpu.VMEM((_CHUNK,), jnp.int32),
            pltpu.VMEM((_CHUNK,), jnp.int32),
            pltpu.VMEM((_CHUNK, PAIR_DIM), jnp.float32),
            pltpu.VMEM((_CHUNK, PAIR_DIM), jnp.float32),
            pltpu.VMEM((_CHUNK, OUT_DIM), jnp.float32),
            pltpu.VMEM((svocab * SHAPE_DIM,), jnp.float32),
            pltpu.SemaphoreType.DMA,
            pltpu.SemaphoreType.DMA,
            pltpu.SemaphoreType.DMA,
            pltpu.SemaphoreType.DMA,
        ],
    )(_make_body(num_chunks_total, svocab))
    return call(whalf, p64, sidx, wtab2, stab_flat)


# speedup vs baseline: 2.3806x; 1.0016x over previous
import functools

import jax
import jax.numpy as jnp
from jax import lax
from jax.experimental import pallas as pl
from jax.experimental.layout import Format, Layout
from jax.experimental.pallas import tpu as pltpu
from jax.experimental.pallas import tpu_sc as plsc

WORD_DIM = 64
SHAPE_DIM = 16
OUT_DIM = WORD_DIM + SHAPE_DIM
PAIR_DIM = 2 * WORD_DIM

_NUM_CORES = 2
_NUM_SUBCORES = 16
_NUM_WORKERS = _NUM_CORES * _NUM_SUBCORES

_IDX_W = 128
_CHUNK = 256
_NIDX = _CHUNK // _IDX_W
_L = 16


def _make_body(num_chunks_total, shape_vocab):
    num_chunks = num_chunks_total // _NUM_WORKERS
    assert num_chunks % 2 == 0

    def body(whalf_hbm, p64_hbm, sidx_hbm, wtab_raw, stab_hbm, out_raw,
             wh0, wh1, p0, p1, s0, s1, pair0, pair1, comb0, stab_v,
             g0, g1, w0, w1):
        wtab_hbm = wtab_raw
        out_hbm = out_raw.reshape(num_chunks_total * _CHUNK, OUT_DIM)
        wh = (wh0, wh1)
        pp = (p0, p1)
        ss = (s0, s1)
        pair = (pair0, pair1)
        comb = (comb0, comb0)
        gsem = (g0, g1)
        wsem = (w0, w0)
        sid = lax.axis_index("s")
        wid = sid * _NUM_CORES + lax.axis_index("c")
        chunk0 = wid * num_chunks

        pltpu.sync_copy(stab_hbm, stab_v)
        iota = lax.iota(jnp.int32, _L)

        def fire(ci, b):
            pltpu.sync_copy(whalf_hbm.at[ci], wh[b])
            pltpu.async_copy(p64_hbm.at[ci], pp[b], gsem[b])
            pltpu.async_copy(sidx_hbm.at[ci], ss[b], gsem[b])
            for j in range(_NIDX):
                pltpu.async_copy(
                    wtab_hbm.at[wh[b].at[j]],
                    pair[b].at[pl.ds(j * _IDX_W, _IDX_W)], gsem[b])

        def gather_wait(ci, b):
            pltpu.make_async_copy(p64_hbm.at[ci], pp[b], gsem[b]).wait()
            pltpu.make_async_copy(sidx_hbm.at[ci], ss[b], gsem[b]).wait()
            for j in range(_NIDX):
                pltpu.make_async_copy(
                    wtab_hbm.at[wh[b].at[j]],
                    pair[b].at[pl.ds(j * _IDX_W, _IDX_W)], gsem[b]).wait()

        def write_wait(ci, b):
            pltpu.make_async_copy(
                comb[b], out_hbm.at[pl.ds(ci * _CHUNK, _CHUNK)],
                wsem[b]).wait()

        def select(b):
            def step(t, carry2):
                rowv = t * _L + iota
                src = pp[b][pl.ds(t * _L, _L)]
                sv = ss[b][pl.ds(t * _L, _L)] * SHAPE_DIM
                for c in range(WORD_DIM):
                    col = (iota + c) & (WORD_DIM - 1)
                    flat = src + col
                    val = plsc.load_gather(pair[b], [rowv, flat])
                    plsc.store_scatter(comb[b], [rowv, col], val)
                for c in range(SHAPE_DIM):
                    col = (iota + c) & (SHAPE_DIM - 1)
                    val = plsc.load_gather(stab_v, [sv + col])
                    plsc.store_scatter(comb[b], [rowv, WORD_DIM + col], val)
                return carry2

            lax.fori_loop(0, _CHUNK // _L, step, 0)

        fire(chunk0, 0)

        def pair_iter(g, carry):
            ci0 = chunk0 + 2 * g
            for b in range(2):
                ci = ci0 + b
                nxt = ci + 1

                @pl.when(nxt < chunk0 + num_chunks)
                def _():
                    fire(nxt, 1 - b)

                gather_wait(ci, b)

                @pl.when(ci > chunk0)
                def _():
                    write_wait(ci - 1, b)

                select(b)
                pltpu.async_copy(
                    comb[b], out_hbm.at[pl.ds(ci * _CHUNK, _CHUNK)], wsem[b])
            return carry

        lax.fori_loop(0, num_chunks // 2, pair_iter, 0)
        write_wait(chunk0 + num_chunks - 1, 1)

    return body


def _kernel_impl(word_id, shape_id, word_table, shape_table):
    b, s = word_id.shape
    n = b * s
    num_chunks_total = n // _CHUNK
    wvocab = word_table.shape[0]
    svocab = shape_table.shape[0]

    wi = word_id.astype(jnp.int32)
    whalf = (wi >> 1).reshape(num_chunks_total, _NIDX, _IDX_W)
    p64 = ((wi & 1) << 6).reshape(num_chunks_total, _CHUNK)
    sidx = shape_id.astype(jnp.int32).reshape(num_chunks_total, _CHUNK)
    stab_flat = shape_table.reshape(svocab * SHAPE_DIM)
    wtab2 = word_table.reshape(wvocab // 2, PAIR_DIM)

    call = functools.partial(
        pl.kernel,
        out_type=jax.ShapeDtypeStruct((b, s, OUT_DIM), jnp.float32),
        mesh=plsc.VectorSubcoreMesh(core_axis_name="c", subcore_axis_name="s"),
        compiler_params=pltpu.CompilerParams(needs_layout_passes=False),
        scratch_types=[
            pltpu.VMEM((_NIDX, _IDX_W), jnp.int32),
            pltpu.VMEM((_NIDX, _IDX_W), jnp.int32),
            pltpu.VMEM((_CHUNK,), jnp.int32),
            pltpu.VMEM((_CHUNK,), jnp.int32),
            pltpu.VMEM((_CHUNK,), jnp.int32),
            pltpu.VMEM((_CHUNK,), jnp.int32),
            pltpu.VMEM((_CHUNK, PAIR_DIM), jnp.float32),
            pltpu.VMEM((_CHUNK, PAIR_DIM), jnp.float32),
            pltpu.VMEM((_CHUNK, OUT_DIM), jnp.float32),
            pltpu.VMEM((svocab * SHAPE_DIM,), jnp.float32),
            pltpu.SemaphoreType.DMA,
            pltpu.SemaphoreType.DMA,
            pltpu.SemaphoreType.DMA,
            pltpu.SemaphoreType.DMA,
        ],
    )(_make_body(num_chunks_total, svocab))
    return call(whalf, p64, sidx, wtab2, stab_flat)


@functools.lru_cache(maxsize=1)
def _jitted_kernel():
    try:
        dev = jax.devices("tpu")[0]
        fmt = Format(Layout(major_to_minor=(0, 1, 2), tiling=((8,),)),
                     jax.sharding.SingleDeviceSharding(dev))
        return jax.jit(_kernel_impl, out_shardings=fmt)
    except Exception:
        return jax.jit(_kernel_impl)


def kernel(word_id, shape_id, word_table, shape_table):
    return _jitted_kernel()(word_id, shape_id, word_table, shape_table)


# speedup vs baseline: 2.8444x; 1.1948x over previous
import functools

import jax
import jax.numpy as jnp
from jax import lax
from jax.experimental import pallas as pl
from jax.experimental.pallas import tpu as pltpu
from jax.experimental.pallas import tpu_sc as plsc

WORD_DIM = 64
SHAPE_DIM = 16
OUT_DIM = WORD_DIM + SHAPE_DIM
PAIR_DIM = 2 * WORD_DIM

_NUM_CORES = 2
_NUM_SUBCORES = 16
_NUM_WORKERS = _NUM_CORES * _NUM_SUBCORES

_IDX_W = 128
_CHUNK = 256
_NIDX = _CHUNK // _IDX_W
_L = 16


def _make_body(batch, seq):
    blocks_per_seq = batch // _CHUNK
    num_chunks_total = seq * blocks_per_seq
    num_chunks = num_chunks_total // _NUM_WORKERS
    assert num_chunks % 2 == 0

    def body(whalf_hbm, p64_hbm, sidx_hbm, wtab_hbm, stab_hbm, out_hbm,
             wh0, wh1, p0, p1, s0, s1, pair0, pair1, comb_t, stab_v,
             g0, g1, wsem):
        wh = (wh0, wh1)
        pp = (p0, p1)
        ss = (s0, s1)
        pair = (pair0, pair1)
        gsem = (g0, g1)
        sid = lax.axis_index("s")
        wid = sid * _NUM_CORES + lax.axis_index("c")
        chunk0 = wid * num_chunks

        pltpu.sync_copy(stab_hbm, stab_v)
        iota = lax.iota(jnp.int32, _L)

        def out_dst(ci):
            s_pos = ci // blocks_per_seq
            blk = ci % blocks_per_seq
            return out_hbm.at[s_pos, :, pl.ds(blk * _CHUNK, _CHUNK)]

        def fire(ci, b):
            pltpu.sync_copy(whalf_hbm.at[ci], wh[b])
            pltpu.async_copy(p64_hbm.at[ci], pp[b], gsem[b])
            pltpu.async_copy(sidx_hbm.at[ci], ss[b], gsem[b])
            for j in range(_NIDX):
                pltpu.async_copy(
                    wtab_hbm.at[wh[b].at[j]],
                    pair[b].at[pl.ds(j * _IDX_W, _IDX_W)], gsem[b])

        def gather_wait(ci, b):
            pltpu.make_async_copy(p64_hbm.at[ci], pp[b], gsem[b]).wait()
            pltpu.make_async_copy(sidx_hbm.at[ci], ss[b], gsem[b]).wait()
            for j in range(_NIDX):
                pltpu.make_async_copy(
                    wtab_hbm.at[wh[b].at[j]],
                    pair[b].at[pl.ds(j * _IDX_W, _IDX_W)], gsem[b]).wait()

        def write_wait(ci):
            pltpu.make_async_copy(comb_t, out_dst(ci), wsem).wait()

        def select(b):
            def step(t, carry2):
                rowv = t * _L + iota
                src = pp[b][pl.ds(t * _L, _L)]
                sv = ss[b][pl.ds(t * _L, _L)] * SHAPE_DIM
                for c in range(WORD_DIM):
                    col = (iota + c) & (WORD_DIM - 1)
                    val = plsc.load_gather(pair[b], [rowv, src + col])
                    plsc.store_scatter(comb_t, [col, rowv], val)
                for c in range(SHAPE_DIM):
                    col = (iota + c) & (SHAPE_DIM - 1)
                    val = plsc.load_gather(stab_v, [sv + col])
                    plsc.store_scatter(comb_t, [WORD_DIM + col, rowv], val)
                return carry2

            lax.fori_loop(0, _CHUNK // _L, step, 0)

        fire(chunk0, 0)

        def pair_iter(g, carry):
            ci0 = chunk0 + 2 * g
            for b in range(2):
                ci = ci0 + b
                nxt = ci + 1

                @pl.when(nxt < chunk0 + num_chunks)
                def _():
                    fire(nxt, 1 - b)

                gather_wait(ci, b)

                @pl.when(ci > chunk0)
                def _():
                    write_wait(ci - 1)

                select(b)
                pltpu.async_copy(comb_t, out_dst(ci), wsem)
            return carry

        lax.fori_loop(0, num_chunks // 2, pair_iter, 0)
        write_wait(chunk0 + num_chunks - 1)

    return body


@jax.jit
def kernel(word_id, shape_id, word_table, shape_table):
    b, s = word_id.shape
    num_chunks_total = (b * s) // _CHUNK
    wvocab = word_table.shape[0]
    svocab = shape_table.shape[0]

    wi_t = word_id.T.astype(jnp.int32)
    si_t = shape_id.T.astype(jnp.int32)
    whalf = (wi_t >> 1).reshape(num_chunks_total, _NIDX, _IDX_W)
    p64 = ((wi_t & 1) << 6).reshape(num_chunks_total, _CHUNK)
    sidx = si_t.reshape(num_chunks_total, _CHUNK)
    stab_flat = shape_table.reshape(svocab * SHAPE_DIM)
    wtab2 = word_table.reshape(wvocab // 2, PAIR_DIM)

    call = functools.partial(
        pl.kernel,
        out_type=jax.ShapeDtypeStruct((s, OUT_DIM, b), jnp.float32),
        mesh=plsc.VectorSubcoreMesh(core_axis_name="c", subcore_axis_name="s"),
        compiler_params=pltpu.CompilerParams(needs_layout_passes=False),
        scratch_types=[
            pltpu.VMEM((_NIDX, _IDX_W), jnp.int32),
            pltpu.VMEM((_NIDX, _IDX_W), jnp.int32),
            pltpu.VMEM((_CHUNK,), jnp.int32),
            pltpu.VMEM((_CHUNK,), jnp.int32),
            pltpu.VMEM((_CHUNK,), jnp.int32),
            pltpu.VMEM((_CHUNK,), jnp.int32),
            pltpu.VMEM((_CHUNK, PAIR_DIM), jnp.float32),
            pltpu.VMEM((_CHUNK, PAIR_DIM), jnp.float32),
            pltpu.VMEM((OUT_DIM, _CHUNK), jnp.float32),
            pltpu.VMEM((svocab * SHAPE_DIM,), jnp.float32),
            pltpu.SemaphoreType.DMA,
            pltpu.SemaphoreType.DMA,
            pltpu.SemaphoreType.DMA,
        ],
    )(_make_body(b, s))
    out_t = call(whalf, p64, sidx, wtab2, stab_flat)
    return jnp.transpose(out_t, (2, 0, 1))


# speedup vs baseline: 3.1204x; 1.0970x over previous
import functools

import jax
import jax.numpy as jnp
from jax import lax
from jax.experimental import pallas as pl
from jax.experimental.pallas import tpu as pltpu
from jax.experimental.pallas import tpu_sc as plsc

WORD_DIM = 64
SHAPE_DIM = 16
OUT_DIM = WORD_DIM + SHAPE_DIM
PAIR_DIM = 2 * WORD_DIM

_NUM_CORES = 2
_NUM_SUBCORES = 16
_NUM_WORKERS = _NUM_CORES * _NUM_SUBCORES

_IDX_W = 128
_CHUNK = 256
_NIDX = _CHUNK // _IDX_W
_L = 16


def _make_body(batch, seq):
    blocks_per_seq = batch // _CHUNK
    num_chunks_total = seq * blocks_per_seq
    num_chunks = num_chunks_total // _NUM_WORKERS
    assert num_chunks % 2 == 0

    def body(whalf_hbm, p64_hbm, sidx_hbm, wtab_hbm, stab_hbm, out_hbm,
             wh0, wh1, p0, p1, s0, s1, pair0, pair1, comb_t, stab_v,
             g0, g1, wsem):
        wh = (wh0, wh1)
        pp = (p0, p1)
        ss = (s0, s1)
        pair = (pair0, pair1)
        gsem = (g0, g1)
        sid = lax.axis_index("s")
        wid = sid * _NUM_CORES + lax.axis_index("c")
        chunk0 = wid * num_chunks

        pltpu.sync_copy(stab_hbm, stab_v)
        iota = lax.iota(jnp.int32, _L)

        def out_dst(ci):
            s_pos = ci // blocks_per_seq
            blk = ci % blocks_per_seq
            return out_hbm.at[s_pos, :, pl.ds(blk * _CHUNK, _CHUNK)]

        def fire(ci, b):
            pltpu.sync_copy(whalf_hbm.at[ci], wh[b])
            pltpu.async_copy(p64_hbm.at[ci], pp[b], gsem[b])
            pltpu.async_copy(sidx_hbm.at[ci], ss[b], gsem[b])
            for j in range(_NIDX):
                pltpu.async_copy(
                    wtab_hbm.at[wh[b].at[j]],
                    pair[b].at[pl.ds(j * _IDX_W, _IDX_W)], gsem[b])

        def gather_wait(ci, b):
            pltpu.make_async_copy(p64_hbm.at[ci], pp[b], gsem[b]).wait()
            pltpu.make_async_copy(sidx_hbm.at[ci], ss[b], gsem[b]).wait()
            for j in range(_NIDX):
                pltpu.make_async_copy(
                    wtab_hbm.at[wh[b].at[j]],
                    pair[b].at[pl.ds(j * _IDX_W, _IDX_W)], gsem[b]).wait()

        def write_wait(ci):
            pltpu.make_async_copy(comb_t, out_dst(ci), wsem).wait()

        def select(b):
            def step(t, carry2):
                rowv = t * _L + iota
                src = pp[b][pl.ds(t * _L, _L)]
                sv = ss[b][pl.ds(t * _L, _L)] * SHAPE_DIM

                def cgroup(k, colw):
                    for _ in range(_L):
                        val = plsc.load_gather(pair[b], [rowv, src + colw])
                        plsc.store_scatter(comb_t, [colw, rowv], val)
                        colw = (colw + 1) & (WORD_DIM - 1)
                    return colw

                lax.fori_loop(0, WORD_DIM // _L, cgroup, iota)

                def sgroup(k, cols):
                    for _ in range(_L):
                        val = plsc.load_gather(stab_v, [sv + cols])
                        plsc.store_scatter(
                            comb_t, [WORD_DIM + cols, rowv], val)
                        cols = (cols + 1) & (SHAPE_DIM - 1)
                    return cols

                lax.fori_loop(0, 1, sgroup, iota & (SHAPE_DIM - 1))
                return carry2

            lax.fori_loop(0, _CHUNK // _L, step, 0)

        fire(chunk0, 0)

        def pair_iter(g, carry):
            ci0 = chunk0 + 2 * g
            for b in range(2):
                ci = ci0 + b
                nxt = ci + 1

                @pl.when(nxt < chunk0 + num_chunks)
                def _():
                    fire(nxt, 1 - b)

                gather_wait(ci, b)

                @pl.when(ci > chunk0)
                def _():
                    write_wait(ci - 1)

                select(b)
                pltpu.async_copy(comb_t, out_dst(ci), wsem)
            return carry

        lax.fori_loop(0, num_chunks // 2, pair_iter, 0)
        write_wait(chunk0 + num_chunks - 1)

    return body


@jax.jit
def kernel(word_id, shape_id, word_table, shape_table):
    b, s = word_id.shape
    num_chunks_total = (b * s) // _CHUNK
    wvocab = word_table.shape[0]
    svocab = shape_table.shape[0]

    wi_t = word_id.T.astype(jnp.int32)
    si_t = shape_id.T.astype(jnp.int32)
    whalf = (wi_t >> 1).reshape(num_chunks_total, _NIDX, _IDX_W)
    p64 = ((wi_t & 1) << 6).reshape(num_chunks_total, _CHUNK)
    sidx = si_t.reshape(num_chunks_total, _CHUNK)
    stab_flat = shape_table.reshape(svocab * SHAPE_DIM)
    wtab2 = word_table.reshape(wvocab // 2, PAIR_DIM)

    call = functools.partial(
        pl.kernel,
        out_type=jax.ShapeDtypeStruct((s, OUT_DIM, b), jnp.float32),
        mesh=plsc.VectorSubcoreMesh(core_axis_name="c", subcore_axis_name="s"),
        compiler_params=pltpu.CompilerParams(needs_layout_passes=False,
                                             disable_bounds_checks=True),
        scratch_types=[
            pltpu.VMEM((_NIDX, _IDX_W), jnp.int32),
            pltpu.VMEM((_NIDX, _IDX_W), jnp.int32),
            pltpu.VMEM((_CHUNK,), jnp.int32),
            pltpu.VMEM((_CHUNK,), jnp.int32),
            pltpu.VMEM((_CHUNK,), jnp.int32),
            pltpu.VMEM((_CHUNK,), jnp.int32),
            pltpu.VMEM((_CHUNK, PAIR_DIM), jnp.float32),
            pltpu.VMEM((_CHUNK, PAIR_DIM), jnp.float32),
            pltpu.VMEM((OUT_DIM, _CHUNK), jnp.float32),
            pltpu.VMEM((svocab * SHAPE_DIM,), jnp.float32),
            pltpu.SemaphoreType.DMA,
            pltpu.SemaphoreType.DMA,
            pltpu.SemaphoreType.DMA,
        ],
    )(_make_body(b, s))
    out_t = call(whalf, p64, sidx, wtab2, stab_flat)
    return jnp.transpose(out_t, (2, 0, 1))


# speedup vs baseline: 3.1213x; 1.0003x over previous
import functools

import jax
import jax.numpy as jnp
from jax import lax
from jax.experimental import pallas as pl
from jax.experimental.layout import Format, Layout, with_layout_constraint
from jax.experimental.pallas import tpu as pltpu
from jax.experimental.pallas import tpu_sc as plsc

WORD_DIM = 64
SHAPE_DIM = 16
OUT_DIM = WORD_DIM + SHAPE_DIM
PAIR_DIM = 2 * WORD_DIM

_NUM_CORES = 2
_NUM_SUBCORES = 16
_NUM_WORKERS = _NUM_CORES * _NUM_SUBCORES

_IDX_W = 128
_CHUNK = 256
_NIDX = _CHUNK // _IDX_W
_L = 16


def _make_body(batch, seq):
    blocks_per_seq = batch // _CHUNK
    num_chunks_total = seq * blocks_per_seq
    num_chunks = num_chunks_total // _NUM_WORKERS
    assert num_chunks % 2 == 0

    def body(whalf_hbm, p64_hbm, sidx_hbm, wtab_hbm, stab_hbm, out_hbm,
             wh0, wh1, p0, p1, s0, s1, pair0, pair1, comb_t, stab_v,
             g0, g1, wsem):
        wh = (wh0, wh1)
        pp = (p0, p1)
        ss = (s0, s1)
        pair = (pair0, pair1)
        gsem = (g0, g1)
        sid = lax.axis_index("s")
        wid = sid * _NUM_CORES + lax.axis_index("c")
        chunk0 = wid * num_chunks

        pltpu.sync_copy(stab_hbm, stab_v)
        iota = lax.iota(jnp.int32, _L)

        def out_dst(ci):
            s_pos = ci // blocks_per_seq
            blk = ci % blocks_per_seq
            return out_hbm.at[s_pos, :, pl.ds(blk * _CHUNK, _CHUNK)]

        def fire(ci, b):
            pltpu.sync_copy(whalf_hbm.at[ci], wh[b])
            pltpu.async_copy(p64_hbm.at[ci], pp[b], gsem[b])
            pltpu.async_copy(sidx_hbm.at[ci], ss[b], gsem[b])
            for j in range(_NIDX):
                pltpu.async_copy(
                    wtab_hbm.at[wh[b].at[j]],
                    pair[b].at[pl.ds(j * _IDX_W, _IDX_W)], gsem[b])

        def gather_wait(ci, b):
            pltpu.make_async_copy(p64_hbm.at[ci], pp[b], gsem[b]).wait()
            pltpu.make_async_copy(sidx_hbm.at[ci], ss[b], gsem[b]).wait()
            for j in range(_NIDX):
                pltpu.make_async_copy(
                    wtab_hbm.at[wh[b].at[j]],
                    pair[b].at[pl.ds(j * _IDX_W, _IDX_W)], gsem[b]).wait()

        def write_wait(ci):
            pltpu.make_async_copy(comb_t, out_dst(ci), wsem).wait()

        def select(b):
            def step(t, carry2):
                rowv = t * _L + iota
                src = pp[b][pl.ds(t * _L, _L)]
                sv = ss[b][pl.ds(t * _L, _L)] * SHAPE_DIM

                def cgroup(k, colw):
                    for _ in range(_L):
                        val = plsc.load_gather(pair[b], [rowv, src + colw])
                        plsc.store_scatter(comb_t, [colw, rowv], val)
                        colw = (colw + 1) & (WORD_DIM - 1)
                    return colw

                lax.fori_loop(0, WORD_DIM // _L, cgroup, iota)

                def sgroup(k, cols):
                    for _ in range(_L):
                        val = plsc.load_gather(stab_v, [sv + cols])
                        plsc.store_scatter(
                            comb_t, [WORD_DIM + cols, rowv], val)
                        cols = (cols + 1) & (SHAPE_DIM - 1)
                    return cols

                lax.fori_loop(0, 1, sgroup, iota & (SHAPE_DIM - 1))
                return carry2

            lax.fori_loop(0, _CHUNK // _L, step, 0)

        fire(chunk0, 0)

        def pair_iter(g, carry):
            ci0 = chunk0 + 2 * g
            for b in range(2):
                ci = ci0 + b
                nxt = ci + 1

                @pl.when(nxt < chunk0 + num_chunks)
                def _():
                    fire(nxt, 1 - b)

                gather_wait(ci, b)

                @pl.when(ci > chunk0)
                def _():
                    write_wait(ci - 1)

                select(b)
                pltpu.async_copy(comb_t, out_dst(ci), wsem)
            return carry

        lax.fori_loop(0, num_chunks // 2, pair_iter, 0)
        write_wait(chunk0 + num_chunks - 1)

    return body


@jax.jit
def kernel(word_id, shape_id, word_table, shape_table):
    b, s = word_id.shape
    num_chunks_total = (b * s) // _CHUNK
    wvocab = word_table.shape[0]
    svocab = shape_table.shape[0]

    wi_t = word_id.T.astype(jnp.int32)
    si_t = shape_id.T.astype(jnp.int32)
    whalf = (wi_t >> 1).reshape(num_chunks_total, _NIDX, _IDX_W)
    p64 = ((wi_t & 1) << 6).reshape(num_chunks_total, _CHUNK)
    sidx = si_t.reshape(num_chunks_total, _CHUNK)
    stab_flat = shape_table.reshape(svocab * SHAPE_DIM)
    wtab2 = word_table.reshape(wvocab // 2, PAIR_DIM)
    try:
        _dev = jax.devices("tpu")[0]
        wtab2 = with_layout_constraint(
            wtab2,
            Format(Layout(major_to_minor=(0, 1), tiling=((8,),)),
                   jax.sharding.SingleDeviceSharding(_dev)))
    except Exception:
        pass

    call = functools.partial(
        pl.kernel,
        out_type=jax.ShapeDtypeStruct((s, OUT_DIM, b), jnp.float32),
        mesh=plsc.VectorSubcoreMesh(core_axis_name="c", subcore_axis_name="s"),
        compiler_params=pltpu.CompilerParams(needs_layout_passes=False,
                                             disable_bounds_checks=True),
        scratch_types=[
            pltpu.VMEM((_NIDX, _IDX_W), jnp.int32),
            pltpu.VMEM((_NIDX, _IDX_W), jnp.int32),
            pltpu.VMEM((_CHUNK,), jnp.int32),
            pltpu.VMEM((_CHUNK,), jnp.int32),
            pltpu.VMEM((_CHUNK,), jnp.int32),
            pltpu.VMEM((_CHUNK,), jnp.int32),
            pltpu.VMEM((_CHUNK, PAIR_DIM), jnp.float32),
            pltpu.VMEM((_CHUNK, PAIR_DIM), jnp.float32),
            pltpu.VMEM((OUT_DIM, _CHUNK), jnp.float32),
            pltpu.VMEM((svocab * SHAPE_DIM,), jnp.float32),
            pltpu.SemaphoreType.DMA,
            pltpu.SemaphoreType.DMA,
            pltpu.SemaphoreType.DMA,
        ],
    )(_make_body(b, s))
    out_t = call(whalf, p64, sidx, wtab2, stab_flat)
    return jnp.transpose(out_t, (2, 0, 1))
